# baseline (speedup 1.0000x reference)
import functools

import jax
import jax.numpy as jnp
from jax import lax
from jax.experimental import pallas as pl
from jax.experimental.pallas import tpu as pltpu
from jax.experimental.pallas import tpu_sc as plsc

NW = 32
L = 16
NC = 2
SB = 8
H = 32768
S = 2048
LCAP = H + 5 * L


def _sc_body(hist_hbm, t_hbm, w0_hbm, w1_hbm, hv_hbm,
             hist_v, t_v, w0_v, w1_v, list_v, list2_v, qval_v, qidx_v,
             hvout_v, sbstart_v, sbend_v, sem):
    c = lax.axis_index("c")
    s = lax.axis_index("s")
    w = (s * NC + c).astype(jnp.int32)

    pltpu.sync_copy(hist_hbm, hist_v)
    pltpu.sync_copy(t_hbm, t_v)
    pltpu.sync_copy(w0_hbm, w0_v)
    pltpu.sync_copy(w1_hbm, w1_v)

    iota = lax.broadcasted_iota(jnp.int32, (L,), 0)
    ninf = jnp.full((L,), -jnp.inf, jnp.float32)
    pinf = jnp.full((L,), jnp.inf, jnp.float32)
    one_i = jnp.full((L,), 1, jnp.int32)
    lsac_i = jnp.full((L,), LCAP - L, jnp.int32) + iota
    qsac_i = jnp.full((L,), S, jnp.int32) + iota
    big_i = jnp.full((L,), L, jnp.int32)
    zero_i = jnp.zeros((L,), jnp.int32)
    sb1_i = jnp.full((L,), SB - 1, jnp.int32)

    def bcast(a):
        return jnp.broadcast_to(a, (L,))

    wf = w.astype(jnp.float32)
    lo_f = wf * (1.0 / NW)
    lo_fv = bcast(lo_f)
    lov = jnp.where(w == 0, -jnp.inf, lo_f)
    hiv = jnp.where(w == NW - 1, jnp.inf, (wf + 1.0) * (1.0 / NW))
    lovv = bcast(lov)
    hivv = bcast(hiv)
    sb_scale = jnp.full((L,), float(NW * SB), jnp.float32)

    def sub_bucket(vals):
        r = ((vals - lo_fv) * sb_scale).astype(jnp.int32)
        return jnp.minimum(jnp.maximum(r, zero_i), sb1_i)

    def p1_step(i, cnt, p1, p2):
        v = hist_v[pl.ds(i * L, L)]
        bv = jnp.where(v < lovv, v, ninf)
        n1 = jnp.maximum(p1, bv)
        p2 = jnp.maximum(p2, jnp.minimum(p1, bv))
        inr = (v >= lovv) & (v < hivv)
        pos = cnt * L + iota
        plsc.store_scatter(list_v, [pos], v, mask=inr)
        return cnt + inr.astype(jnp.int32), n1, p2

    def p1_body(i, carry):
        cnt, p1, p2 = carry
        for k in range(4):
            cnt, p1, p2 = p1_step(4 * i + k, cnt, p1, p2)
        return cnt, p1, p2

    with jax.named_scope("ph1"):
        cntv, p1, p2 = lax.fori_loop(
            0, H // L // 4, p1_body, (zero_i, ninf, ninf))
        nrows = jnp.max(cntv)

    subm1 = ninf
    subm2 = ninf
    startv = zero_i
    endv = zero_i
    rtop = jnp.int32(0)
    _scope1b = jax.named_scope("ph1b"); _scope1b.__enter__()
    for sb in range(SB):
        sbv = jnp.full((L,), sb, jnp.int32)
        lane_is_sb = iota == sbv
        startv = jnp.where(lane_is_sb, bcast(rtop), startv)

        def rp_step(j, rtv, a1, a2, sbv=sbv):
            lv = list_v[pl.ds(j * L, L)]
            m = (bcast(j) < cntv) & (sub_bucket(lv) == sbv)
            g = jnp.where(m, lv, ninf)
            n1 = jnp.maximum(a1, g)
            a2 = jnp.maximum(a2, jnp.minimum(a1, g))
            cum = plsc.cumsum(m.astype(jnp.int32))
            pos = jnp.where(m, rtv + cum - one_i, lsac_i)
            plsc.store_scatter(list2_v, [pos], lv)
            return rtv + plsc.all_reduce_population_count(m), n1, a2

        def rp_body(j, carry, rp_step=rp_step):
            rt, a1, a2 = carry
            rtv = bcast(rt)
            rtv, a1, a2 = rp_step(2 * j, rtv, a1, a2)
            rtv, a1, a2 = rp_step(2 * j + 1, rtv, a1, a2)
            return jnp.max(rtv), a1, a2

        rtop, a1, a2 = lax.fori_loop(0, (nrows + 1) // 2, rp_body,
                                     (rtop, ninf, ninf))
        m1s = jnp.max(a1)
        fiv = bcast(jnp.min(jnp.where(a1 == bcast(m1s), iota, big_i)))
        first = iota == fiv
        sm = jnp.max(jnp.where(first, ninf, a1))
        m2c = jnp.max(jnp.where(first, a2, ninf))
        m2s = jnp.maximum(sm, m2c)
        subm1 = jnp.where(lane_is_sb, bcast(m1s), subm1)
        subm2 = jnp.where(lane_is_sb, bcast(m2s), subm2)
        endv = jnp.where(lane_is_sb, bcast(rtop), endv)
    _scope1b.__exit__(None, None, None)
    sbstart_v[pl.ds(0, L)] = startv
    sbend_v[pl.ds(0, L)] = endv
    sbstart_v[pl.ds(L, L)] = zero_i
    sbend_v[pl.ds(L, L)] = zero_i

    def p2_body(i, qoffv):
        tv = t_v[pl.ds(i * L, L)]
        qm = (tv >= lovv) & (tv < hivv)
        cum = plsc.cumsum(qm.astype(jnp.int32))
        pos = jnp.where(qm, qoffv + cum - one_i, qsac_i)
        plsc.store_scatter(qval_v, [pos], tv)
        plsc.store_scatter(qidx_v, [pos], iota + bcast(i * L))
        return qoffv + plsc.all_reduce_population_count(qm)

    with jax.named_scope("ph2"):
        qcnt = jnp.max(lax.fori_loop(0, S // L, p2_body, zero_i))

    w0v = w0_v[...]
    w1v = w1_v[...]
    lcapv = jnp.full((L,), LCAP - 1, jnp.int32)

    sac_spread = jnp.full((L,), LCAP - L, jnp.int32) + iota

    def q_body(q, carry):
        tq = qval_v[pl.ds(q, L)][0]
        tqv = bcast(tq)
        qsb_r = ((tq - lo_f) * float(NW * SB)).astype(jnp.int32)
        qsb = jnp.minimum(jnp.maximum(qsb_r, 0), SB - 1)
        rs = sbstart_v[pl.ds(qsb, L)][0]
        re = sbend_v[pl.ds(qsb, L)][0]
        nch = (re - rs + L - 1) // L
        rev = bcast(re)
        qsbv = bcast(qsb)

        def c_body(j, ac):
            a1, a2 = ac
            idx = bcast(rs + j * L) + iota
            valid = idx < rev
            lv = plsc.load_gather(list2_v, [jnp.minimum(idx, sac_spread)])
            ok = valid & (lv <= tqv)
            g = jnp.where(ok, lv, ninf)
            n1 = jnp.maximum(a1, g)
            a2 = jnp.maximum(a2, jnp.minimum(a1, g))
            return n1, a2

        a1, a2 = lax.fori_loop(0, nch, c_body, (ninf, ninf))
        below_sb = iota < qsbv
        b1 = jnp.where(below_sb, subm1, ninf)
        b2 = jnp.where(below_sb, subm2, ninf)
        c1 = jnp.maximum(a1, b1)
        c2 = jnp.maximum(jnp.minimum(a1, b1), jnp.maximum(a2, b2))
        d1 = jnp.maximum(c1, p1)
        d2 = jnp.maximum(jnp.minimum(c1, p1), jnp.maximum(c2, p2))
        m1v = bcast(jnp.max(d1))
        fiv = bcast(jnp.min(jnp.where(d1 == m1v, iota, big_i)))
        first = iota == fiv
        sm = jnp.max(jnp.where(first, ninf, d1))
        m2c = jnp.max(jnp.where(first, d2, ninf))
        m2v = bcast(jnp.maximum(sm, m2c))
        hvv = w0v * jnp.exp(m1v - tqv) + w1v * jnp.exp(m2v - tqv)
        plsc.store_scatter(hvout_v, [jnp.where(iota == zero_i, bcast(q), qsac_i)],
                           hvv)
        return carry

    with jax.named_scope("ph3"):
        lax.fori_loop(0, qcnt, q_body, jnp.int32(0))

    qcntv = bcast(qcnt)

    def s_body(j, carry):
        idxv = qidx_v[pl.ds(j * L, L)]
        valid = (iota + bcast(j * L)) < qcntv
        idxv = jnp.where(valid, idxv, jnp.full((L,), S, jnp.int32))
        pltpu.async_copy(hvout_v.at[pl.ds(j * L, L)], hv_hbm.at[idxv],
                         sem).wait()
        return carry

    with jax.named_scope("ph4"):
        lax.fori_loop(0, (qcnt + L - 1) // L, s_body, jnp.int32(0))


_sc_kernel = functools.partial(
    pl.kernel,
    out_type=jax.ShapeDtypeStruct((S + L,), jnp.float32),
    mesh=plsc.VectorSubcoreMesh(core_axis_name="c", subcore_axis_name="s"),
    scratch_types=[
        pltpu.VMEM((H,), jnp.float32),
        pltpu.VMEM((S,), jnp.float32),
        pltpu.VMEM((L,), jnp.float32),
        pltpu.VMEM((L,), jnp.float32),
        pltpu.VMEM((LCAP,), jnp.float32),
        pltpu.VMEM((LCAP,), jnp.float32),
        pltpu.VMEM((S + L,), jnp.float32),
        pltpu.VMEM((S + L,), jnp.int32),
        pltpu.VMEM((S + L,), jnp.float32),
        pltpu.VMEM((2 * L,), jnp.int32),
        pltpu.VMEM((2 * L,), jnp.int32),
        pltpu.SemaphoreType.DMA,
    ],
    compiler_params=pltpu.CompilerParams(needs_layout_passes=False),
)(_sc_body)


def _combine_body(hv_ref, x_ref, out_ref):
    out_ref[...] = hv_ref[...] - x_ref[...]


@jax.jit
def kernel(x, t, history, W_hist):
    B = x.shape[0]
    t_row = t[0, :, 0]
    w0 = jnp.full((L,), W_hist[0, 0], jnp.float32)
    w1 = jnp.full((L,), W_hist[0, 1], jnp.float32)
    hv = _sc_kernel(history, t_row, w0, w1)
    hv2d = hv[:S].reshape(1, S)
    out = pl.pallas_call(
        _combine_body,
        out_shape=jax.ShapeDtypeStruct((B, S), jnp.float32),
    )(hv2d, x)
    return out


# speedup vs baseline: 1.0054x; 1.0054x over previous
import functools

import jax
import jax.numpy as jnp
from jax import lax
from jax.experimental import pallas as pl
from jax.experimental.pallas import tpu as pltpu
from jax.experimental.pallas import tpu_sc as plsc

NW = 32
L = 16
NC = 2
SB = 8
H = 32768
S = 2048
LCAP = H + 5 * L


def _sc_body(hist_hbm, t_hbm, w0_hbm, w1_hbm, hv_hbm,
             hist_v, t_v, w0_v, w1_v, list_v, list2_v, qval_v, qidx_v,
             hvout_v, sbstart_v, sbend_v, sem):
    c = lax.axis_index("c")
    s = lax.axis_index("s")
    w = (s * NC + c).astype(jnp.int32)

    with jax.named_scope("ph0"):
        pltpu.sync_copy(hist_hbm, hist_v)
        pltpu.sync_copy(t_hbm, t_v)
        pltpu.sync_copy(w0_hbm, w0_v)
        pltpu.sync_copy(w1_hbm, w1_v)

    iota = lax.broadcasted_iota(jnp.int32, (L,), 0)
    ninf = jnp.full((L,), -jnp.inf, jnp.float32)
    pinf = jnp.full((L,), jnp.inf, jnp.float32)
    one_i = jnp.full((L,), 1, jnp.int32)
    lsac_i = jnp.full((L,), LCAP - L, jnp.int32) + iota
    qsac_i = jnp.full((L,), S, jnp.int32) + iota
    big_i = jnp.full((L,), L, jnp.int32)
    zero_i = jnp.zeros((L,), jnp.int32)
    sb1_i = jnp.full((L,), SB - 1, jnp.int32)

    def bcast(a):
        return jnp.broadcast_to(a, (L,))

    wf = w.astype(jnp.float32)
    lo_f = wf * (1.0 / NW)
    lo_fv = bcast(lo_f)
    lov = jnp.where(w == 0, -jnp.inf, lo_f)
    hiv = jnp.where(w == NW - 1, jnp.inf, (wf + 1.0) * (1.0 / NW))
    lovv = bcast(lov)
    hivv = bcast(hiv)
    sb_scale = jnp.full((L,), float(NW * SB), jnp.float32)

    def sub_bucket(vals):
        r = ((vals - lo_fv) * sb_scale).astype(jnp.int32)
        return jnp.minimum(jnp.maximum(r, zero_i), sb1_i)

    def p1_step(i, cnt, p1, p2):
        v = hist_v[pl.ds(i * L, L)]
        bv = jnp.where(v < lovv, v, ninf)
        n1 = jnp.maximum(p1, bv)
        p2 = jnp.maximum(p2, jnp.minimum(p1, bv))
        inr = (v >= lovv) & (v < hivv)
        pos = cnt * L + iota
        plsc.store_scatter(list_v, [pos], v, mask=inr)
        return cnt + inr.astype(jnp.int32), n1, p2

    def p1_body(i, carry):
        cnt, p1, p2 = carry
        for k in range(4):
            cnt, p1, p2 = p1_step(4 * i + k, cnt, p1, p2)
        return cnt, p1, p2

    with jax.named_scope("ph1"):
        cntv, p1, p2 = lax.fori_loop(
            0, H // L // 4, p1_body, (zero_i, ninf, ninf))
        nrows = jnp.max(cntv)

    subm1 = ninf
    subm2 = ninf
    startv = zero_i
    endv = zero_i
    rtop = jnp.int32(0)
    _scope1b = jax.named_scope("ph1b"); _scope1b.__enter__()
    for sb in range(SB):
        sbv = jnp.full((L,), sb, jnp.int32)
        lane_is_sb = iota == sbv
        startv = jnp.where(lane_is_sb, bcast(rtop), startv)

        def rp_step(j, rtv, a1, a2, sbv=sbv):
            lv = list_v[pl.ds(j * L, L)]
            m = (bcast(j) < cntv) & (sub_bucket(lv) == sbv)
            g = jnp.where(m, lv, ninf)
            n1 = jnp.maximum(a1, g)
            a2 = jnp.maximum(a2, jnp.minimum(a1, g))
            cum = plsc.cumsum(m.astype(jnp.int32))
            pos = jnp.where(m, rtv + cum - one_i, lsac_i)
            plsc.store_scatter(list2_v, [pos], lv)
            return rtv + plsc.all_reduce_population_count(m), n1, a2

        def rp_body(j, carry, rp_step=rp_step):
            rt, a1, a2 = carry
            rtv = bcast(rt)
            rtv, a1, a2 = rp_step(2 * j, rtv, a1, a2)
            rtv, a1, a2 = rp_step(2 * j + 1, rtv, a1, a2)
            return jnp.max(rtv), a1, a2

        rtop, a1, a2 = lax.fori_loop(0, (nrows + 1) // 2, rp_body,
                                     (rtop, ninf, ninf))
        m1s = jnp.max(a1)
        fiv = bcast(jnp.min(jnp.where(a1 == bcast(m1s), iota, big_i)))
        first = iota == fiv
        sm = jnp.max(jnp.where(first, ninf, a1))
        m2c = jnp.max(jnp.where(first, a2, ninf))
        m2s = jnp.maximum(sm, m2c)
        subm1 = jnp.where(lane_is_sb, bcast(m1s), subm1)
        subm2 = jnp.where(lane_is_sb, bcast(m2s), subm2)
        endv = jnp.where(lane_is_sb, bcast(rtop), endv)
    _scope1b.__exit__(None, None, None)
    sbstart_v[pl.ds(0, L)] = startv
    sbend_v[pl.ds(0, L)] = endv
    sbstart_v[pl.ds(L, L)] = zero_i
    sbend_v[pl.ds(L, L)] = zero_i

    def p2_body(i, qoffv):
        tv = t_v[pl.ds(i * L, L)]
        qm = (tv >= lovv) & (tv < hivv)
        cum = plsc.cumsum(qm.astype(jnp.int32))
        pos = jnp.where(qm, qoffv + cum - one_i, qsac_i)
        plsc.store_scatter(qval_v, [pos], tv)
        plsc.store_scatter(qidx_v, [pos], iota + bcast(i * L))
        return qoffv + plsc.all_reduce_population_count(qm)

    with jax.named_scope("ph2"):
        qcnt = jnp.max(lax.fori_loop(0, S // L, p2_body, zero_i))

    w0v = w0_v[...]
    w1v = w1_v[...]
    lcapv = jnp.full((L,), LCAP - 1, jnp.int32)

    sac_spread = jnp.full((L,), LCAP - L, jnp.int32) + iota

    def q_body(q, carry):
        tq = qval_v[pl.ds(q, L)][0]
        tqv = bcast(tq)
        qsb_r = ((tq - lo_f) * float(NW * SB)).astype(jnp.int32)
        qsb = jnp.minimum(jnp.maximum(qsb_r, 0), SB - 1)
        rs = sbstart_v[pl.ds(qsb, L)][0]
        re = sbend_v[pl.ds(qsb, L)][0]
        nch = (re - rs + L - 1) // L
        rev = bcast(re)
        qsbv = bcast(qsb)

        def c_body(j, ac):
            a1, a2 = ac
            idx = bcast(rs + j * L) + iota
            valid = idx < rev
            lv = plsc.load_gather(list2_v, [jnp.minimum(idx, sac_spread)])
            ok = valid & (lv <= tqv)
            g = jnp.where(ok, lv, ninf)
            n1 = jnp.maximum(a1, g)
            a2 = jnp.maximum(a2, jnp.minimum(a1, g))
            return n1, a2

        a1, a2 = lax.fori_loop(0, nch, c_body, (ninf, ninf))
        below_sb = iota < qsbv
        b1 = jnp.where(below_sb, subm1, ninf)
        b2 = jnp.where(below_sb, subm2, ninf)
        c1 = jnp.maximum(a1, b1)
        c2 = jnp.maximum(jnp.minimum(a1, b1), jnp.maximum(a2, b2))
        d1 = jnp.maximum(c1, p1)
        d2 = jnp.maximum(jnp.minimum(c1, p1), jnp.maximum(c2, p2))
        m1v = bcast(jnp.max(d1))
        fiv = bcast(jnp.min(jnp.where(d1 == m1v, iota, big_i)))
        first = iota == fiv
        sm = jnp.max(jnp.where(first, ninf, d1))
        m2c = jnp.max(jnp.where(first, d2, ninf))
        m2v = bcast(jnp.maximum(sm, m2c))
        hvv = w0v * jnp.exp(m1v - tqv) + w1v * jnp.exp(m2v - tqv)
        plsc.store_scatter(hvout_v, [jnp.where(iota == zero_i, bcast(q), qsac_i)],
                           hvv)
        return carry

    with jax.named_scope("ph3"):
        lax.fori_loop(0, qcnt, q_body, jnp.int32(0))

    qcntv = bcast(qcnt)

    def s_body(j, carry):
        idxv = qidx_v[pl.ds(j * L, L)]
        valid = (iota + bcast(j * L)) < qcntv
        idxv = jnp.where(valid, idxv, jnp.full((L,), S, jnp.int32))
        pltpu.async_copy(hvout_v.at[pl.ds(j * L, L)], hv_hbm.at[idxv],
                         sem).wait()
        return carry

    with jax.named_scope("ph4"):
        lax.fori_loop(0, (qcnt + L - 1) // L, s_body, jnp.int32(0))


_sc_kernel = functools.partial(
    pl.kernel,
    out_type=jax.ShapeDtypeStruct((S + L,), jnp.float32),
    mesh=plsc.VectorSubcoreMesh(core_axis_name="c", subcore_axis_name="s"),
    scratch_types=[
        pltpu.VMEM((H,), jnp.float32),
        pltpu.VMEM((S,), jnp.float32),
        pltpu.VMEM((L,), jnp.float32),
        pltpu.VMEM((L,), jnp.float32),
        pltpu.VMEM((LCAP,), jnp.float32),
        pltpu.VMEM((LCAP,), jnp.float32),
        pltpu.VMEM((S + L,), jnp.float32),
        pltpu.VMEM((S + L,), jnp.int32),
        pltpu.VMEM((S + L,), jnp.float32),
        pltpu.VMEM((2 * L,), jnp.int32),
        pltpu.VMEM((2 * L,), jnp.int32),
        pltpu.SemaphoreType.DMA,
    ],
    compiler_params=pltpu.CompilerParams(needs_layout_passes=False),
)(_sc_body)


def _combine_body(hv_ref, x_ref, out_ref):
    out_ref[...] = hv_ref[...] - x_ref[...]


@jax.jit
def kernel(x, t, history, W_hist):
    B = x.shape[0]
    t_row = t[0, :, 0]
    w0 = jnp.full((L,), W_hist[0, 0], jnp.float32)
    w1 = jnp.full((L,), W_hist[0, 1], jnp.float32)
    hv = _sc_kernel(history, t_row, w0, w1)
    hv2d = hv[:S].reshape(1, S)
    out = pl.pallas_call(
        _combine_body,
        out_shape=jax.ShapeDtypeStruct((B, S), jnp.float32),
    )(hv2d, x)
    return out
